# baseline (speedup 1.0000x reference)
import dataclasses
import functools

import jax
import jax.numpy as jnp
from jax import lax
from jax.experimental import pallas as pl
from jax.experimental.pallas import tpu as pltpu
from jax.experimental.pallas import tpu_sc as plsc

NC = 2
NS = 16
NW = NC * NS
LANES = 16
K = 96
NBUF = 3
N_STAGES = 4
ACC_BLK = 128


def _acc_rows(n_nodes):
  return -(-n_nodes // (NS * ACC_BLK)) * NS * ACC_BLK


def _sc_partials(x, src, dst, val, n_chunks, n_nodes, d_feat):
  n_pad = _acc_rows(n_nodes)
  rows_per_tile = n_pad // NS
  copies = []
  off = 0
  while off < rows_per_tile:
    sz = min(K, rows_per_tile - off)
    copies.append((off, sz))
    off += sz
  n_group = n_chunks // N_STAGES
  n_tri = n_group // NBUF
  mesh = plsc.VectorSubcoreMesh(
      core_axis_name="c", subcore_axis_name="s", num_cores=NC,
      num_subcores=NS)
  cp = pltpu.CompilerParams()
  if "needs_layout_passes" in pltpu.CompilerParams.__dataclass_fields__:
    cp = dataclasses.replace(cp, needs_layout_passes=False)

  @functools.partial(
      pl.kernel,
      compiler_params=cp,
      out_type=jax.ShapeDtypeStruct((NC, n_pad, d_feat), jnp.float32),
      mesh=mesh,
      scratch_types=[
          pltpu.VMEM_SHARED((n_pad, d_feat), jnp.float32),
          pltpu.VMEM((n_group, K), jnp.int32),
          pltpu.VMEM((n_group, K), jnp.int32),
          pltpu.VMEM((n_group, K), jnp.float32),
          [pltpu.VMEM((K, d_feat), jnp.float32)] * NBUF,
          [pltpu.SemaphoreType.DMA] * NBUF,
          [pltpu.SemaphoreType.DMA] * NBUF,
      ],
  )
  def sc_kernel(x_hbm, src_hbm, dst_hbm, val_hbm, part_hbm,
                acc, src_v, dst_v, val_v, bufs, gsems, ssems):
    c = lax.axis_index("c")
    s = lax.axis_index("s")
    wid = s * NC + c

    def stage_idx(stage):
      widx = wid * N_STAGES + stage
      pltpu.sync_copy(src_hbm.at[widx], src_v)
      pltpu.sync_copy(dst_hbm.at[widx], dst_v)
      pltpu.sync_copy(val_hbm.at[widx], val_v)

    def start_gather(j, b):
      pltpu.async_copy(x_hbm.at[src_v.at[j]], bufs[b], gsems[b])

    def wait_gather(j, b):
      pltpu.make_async_copy(x_hbm.at[src_v.at[j]], bufs[b], gsems[b]).wait()

    def start_scatter(j, b):
      pltpu.async_copy(bufs[b], acc.at[dst_v.at[j]], ssems[b], add=True)

    def wait_scatter(j, b):
      pltpu.make_async_copy(bufs[b], acc.at[dst_v.at[j]], ssems[b]).wait()

    def scale(j, b):
      jvec = jnp.full((LANES,), j, jnp.int32)
      buf = bufs[b]

      @plsc.parallel_loop(0, K, unroll=8)
      def _(e):
        vsplat = plsc.load_gather(
            val_v, [jvec, jnp.full((LANES,), e, jnp.int32)])
        for t in range(d_feat // LANES):
          sl = buf.at[e, pl.ds(t * LANES, LANES)]
          sl[...] = sl[...] * vsplat

    @pl.loop(0, K)
    def _(i):
      for t in range(d_feat // LANES):
        bufs[0].at[i, pl.ds(t * LANES, LANES)][...] = jnp.zeros(
            (LANES,), jnp.float32)

    for off, sz in copies:
      pltpu.sync_copy(
          bufs[0].at[pl.ds(0, sz)],
          acc.at[pl.ds(s * rows_per_tile + off, sz)])

    for stage in range(N_STAGES):
      stage_idx(stage)
      start_gather(0, 0)
      start_gather(1, 1)
      if stage == 0:
        plsc.subcore_barrier()

      @pl.loop(0, n_tri)
      def _(t):
        for b in range(NBUF):
          j = NBUF * t + b
          bp = (b + NBUF - 1) % NBUF

          def drain_and_prefetch(j=j, bp=bp, first=(b == 0)):
            if first:
              @pl.when(t >= 1)
              def _():
                wait_scatter(j - 1, bp)
            else:
              wait_scatter(j - 1, bp)

          drain_and_prefetch()
          if b == 0:
            start_gather(j + 2, bp)
          else:
            @pl.when(t < n_tri - 1)
            def _():
              start_gather(j + 2, bp)
          wait_gather(j, b)
          scale(j, b)
          start_scatter(j, b)

      wait_scatter(n_group - 1, (n_group - 1) % NBUF)

    plsc.subcore_barrier()

    for off, sz in copies:
      r0 = s * rows_per_tile + off
      pltpu.sync_copy(acc.at[pl.ds(r0, sz)],
                      part_hbm.at[c, pl.ds(r0, sz)])

  return sc_kernel(x, src, dst, val)


def _tc_merge(partials, n_nodes, d_feat):
  def body(p_ref, o_ref):
    o_ref[...] = p_ref[0, :n_nodes] + p_ref[1, :n_nodes]

  return pl.pallas_call(
      body,
      out_shape=jax.ShapeDtypeStruct((n_nodes, d_feat), jnp.float32),
  )(partials)


@jax.jit
def _run(x, adj_indices, adj_values):
  n_nodes, d_feat = x.shape
  n_edges = adj_values.shape[0]
  dst = adj_indices[0].astype(jnp.int32)
  src = adj_indices[1].astype(jnp.int32)
  val = adj_values.astype(jnp.float32)

  m = N_STAGES * NBUF
  n_chunks = -(-n_edges // (NW * K * m)) * m
  e_pad = NW * n_chunks * K
  pad = e_pad - n_edges
  acc_spare = _acc_rows(n_nodes) - n_nodes
  if acc_spare > 0:
    pad_dst = n_nodes + (jnp.arange(pad, dtype=jnp.int32) % acc_spare)
  else:
    pad_dst = jnp.arange(pad, dtype=jnp.int32) % n_nodes
  pad_src = jnp.arange(pad, dtype=jnp.int32) % n_nodes

  def shard(a):
    return a.reshape(NW * N_STAGES, n_chunks // N_STAGES, K)

  src = shard(jnp.concatenate([src, pad_src]))
  dst = shard(jnp.concatenate([dst, pad_dst]))
  val = shard(jnp.pad(val, (0, pad)))

  partials = _sc_partials(x, src, dst, val, n_chunks, n_nodes, d_feat)
  return _tc_merge(partials, n_nodes, d_feat)


def kernel(x, adj_indices, adj_values):
  return _run(x, adj_indices, adj_values)


# speedup vs baseline: 1.0102x; 1.0102x over previous
import dataclasses
import functools

import numpy as np
import jax
import jax.numpy as jnp
from jax import lax
from jax.experimental import pallas as pl
from jax.experimental.pallas import tpu as pltpu
from jax.experimental.pallas import tpu_sc as plsc

NC = 2
NS = 16
NW = NC * NS
LANES = 16
K = 96
NBUF = 3
N_STAGES = 4
ACC_BLK = 128


def _acc_rows(n_nodes):
  return -(-n_nodes // (NS * ACC_BLK)) * NS * ACC_BLK


def _sc_partials(x, src, dst, val, n_chunks, n_nodes, d_feat):
  n_pad = _acc_rows(n_nodes)
  rows_per_tile = n_pad // NS
  copies = []
  off = 0
  while off < rows_per_tile:
    sz = min(K, rows_per_tile - off)
    copies.append((off, sz))
    off += sz
  n_group = n_chunks // N_STAGES
  n_tri = n_group // NBUF
  mesh = plsc.VectorSubcoreMesh(
      core_axis_name="c", subcore_axis_name="s", num_cores=NC,
      num_subcores=NS)
  cp = pltpu.CompilerParams()
  if "needs_layout_passes" in pltpu.CompilerParams.__dataclass_fields__:
    cp = dataclasses.replace(cp, needs_layout_passes=False)

  @functools.partial(
      pl.kernel,
      compiler_params=cp,
      out_type=jax.ShapeDtypeStruct((NC, n_pad, d_feat), jnp.float32),
      mesh=mesh,
      scratch_types=[
          pltpu.VMEM_SHARED((n_pad, d_feat), jnp.float32),
          pltpu.VMEM((n_group, K), jnp.int32),
          pltpu.VMEM((n_group, K), jnp.int32),
          pltpu.VMEM((n_group, K), jnp.float32),
          [pltpu.VMEM((K, d_feat), jnp.float32)] * NBUF,
          [pltpu.SemaphoreType.DMA] * NBUF,
          [pltpu.SemaphoreType.DMA] * NBUF,
      ],
  )
  def sc_kernel(x_hbm, src_hbm, dst_hbm, val_hbm, part_hbm,
                acc, src_v, dst_v, val_v, bufs, gsems, ssems):
    c = lax.axis_index("c")
    s = lax.axis_index("s")
    wid = s * NC + c

    def stage_idx(stage):
      widx = wid * N_STAGES + stage
      pltpu.sync_copy(src_hbm.at[widx], src_v)
      pltpu.sync_copy(dst_hbm.at[widx], dst_v)
      pltpu.sync_copy(val_hbm.at[widx], val_v)

    def start_gather(j, b):
      pltpu.async_copy(x_hbm.at[src_v.at[j]], bufs[b], gsems[b])

    def wait_gather(j, b):
      pltpu.make_async_copy(x_hbm.at[src_v.at[j]], bufs[b], gsems[b]).wait()

    def start_scatter(j, b):
      pltpu.async_copy(bufs[b], acc.at[dst_v.at[j]], ssems[b], add=True)

    def wait_scatter(j, b):
      pltpu.make_async_copy(bufs[b], acc.at[dst_v.at[j]], ssems[b]).wait()

    def scale(j, b):
      jvec = jnp.full((LANES,), j, jnp.int32)
      buf = bufs[b]

      @plsc.parallel_loop(0, K, unroll=4)
      def _(e):
        vsplat = plsc.load_gather(
            val_v, [jvec, jnp.full((LANES,), e, jnp.int32)])
        for t in range(d_feat // LANES):
          sl = buf.at[e, pl.ds(t * LANES, LANES)]
          sl[...] = sl[...] * vsplat

    @pl.loop(0, K)
    def _(i):
      for t in range(d_feat // LANES):
        bufs[0].at[i, pl.ds(t * LANES, LANES)][...] = jnp.zeros(
            (LANES,), jnp.float32)

    for off, sz in copies:
      pltpu.sync_copy(
          bufs[0].at[pl.ds(0, sz)],
          acc.at[pl.ds(s * rows_per_tile + off, sz)])

    for stage in range(N_STAGES):
      stage_idx(stage)
      start_gather(0, 0)
      start_gather(1, 1)
      if stage == 0:
        plsc.subcore_barrier()

      @pl.loop(0, n_tri)
      def _(t):
        for b in range(NBUF):
          j = NBUF * t + b
          bp = (b + NBUF - 1) % NBUF

          def drain_and_prefetch(j=j, bp=bp, first=(b == 0)):
            if first:
              @pl.when(t >= 1)
              def _():
                wait_scatter(j - 1, bp)
            else:
              wait_scatter(j - 1, bp)

          drain_and_prefetch()
          if b == 0:
            start_gather(j + 2, bp)
          else:
            @pl.when(t < n_tri - 1)
            def _():
              start_gather(j + 2, bp)
          wait_gather(j, b)
          scale(j, b)
          start_scatter(j, b)

      wait_scatter(n_group - 1, (n_group - 1) % NBUF)

    plsc.subcore_barrier()

    for off, sz in copies:
      r0 = s * rows_per_tile + off
      pltpu.sync_copy(acc.at[pl.ds(r0, sz)],
                      part_hbm.at[c, pl.ds(r0, sz)])

  return sc_kernel(x, src, dst, val)


def _tc_merge(partials, n_nodes, d_feat):
  def body(p_ref, o_ref):
    o_ref[...] = p_ref[0, :n_nodes] + p_ref[1, :n_nodes]

  return pl.pallas_call(
      body,
      out_shape=jax.ShapeDtypeStruct((n_nodes, d_feat), jnp.float32),
  )(partials)


@jax.jit
def _run(x, adj_indices, adj_values):
  n_nodes, d_feat = x.shape
  n_edges = adj_values.shape[0]
  dst = adj_indices[0].astype(jnp.int32)
  src = adj_indices[1].astype(jnp.int32)
  val = adj_values.astype(jnp.float32)

  m = N_STAGES * NBUF
  n_chunks = -(-n_edges // (NW * K * m)) * m
  e_pad = NW * n_chunks * K
  pad = e_pad - n_edges
  acc_spare = _acc_rows(n_nodes) - n_nodes
  if acc_spare > 0:
    pad_dst = jnp.asarray(
        n_nodes + (np.arange(pad) % acc_spare), dtype=jnp.int32)
  else:
    pad_dst = jnp.asarray(np.arange(pad) % n_nodes, dtype=jnp.int32)
  pad_src = jnp.asarray(np.arange(pad) % n_nodes, dtype=jnp.int32)

  def shard(a):
    return a.reshape(NW * N_STAGES, n_chunks // N_STAGES, K)

  src = shard(jnp.concatenate([src, pad_src]))
  dst = shard(jnp.concatenate([dst, pad_dst]))
  val = shard(jnp.pad(val, (0, pad)))

  partials = _sc_partials(x, src, dst, val, n_chunks, n_nodes, d_feat)
  return _tc_merge(partials, n_nodes, d_feat)


def kernel(x, adj_indices, adj_values):
  return _run(x, adj_indices, adj_values)


# speedup vs baseline: 1.0210x; 1.0106x over previous
import dataclasses
import functools

import numpy as np
import jax
import jax.numpy as jnp
from jax import lax
from jax.experimental import pallas as pl
from jax.experimental.pallas import tpu as pltpu
from jax.experimental.pallas import tpu_sc as plsc

NC = 2
NS = 16
NW = NC * NS
LANES = 16
K = 96
NBUF = 3
N_STAGES = 4
ACC_BLK = 128


def _acc_rows(n_nodes):
  return -(-n_nodes // (NS * ACC_BLK)) * NS * ACC_BLK


def _sc_partials(x, src, dst, val, n_chunks, n_nodes, d_feat):
  n_pad = _acc_rows(n_nodes)
  rows_per_tile = n_pad // NS
  copies = []
  off = 0
  while off < rows_per_tile:
    sz = min(K, rows_per_tile - off)
    copies.append((off, sz))
    off += sz
  n_group = n_chunks // N_STAGES
  n_tri = n_group // NBUF
  mesh = plsc.VectorSubcoreMesh(
      core_axis_name="c", subcore_axis_name="s", num_cores=NC,
      num_subcores=NS)
  cp = pltpu.CompilerParams()
  if "needs_layout_passes" in pltpu.CompilerParams.__dataclass_fields__:
    cp = dataclasses.replace(cp, needs_layout_passes=False)

  @functools.partial(
      pl.kernel,
      compiler_params=cp,
      out_type=jax.ShapeDtypeStruct((NC, n_pad, d_feat), jnp.float32),
      mesh=mesh,
      scratch_types=[
          pltpu.VMEM_SHARED((n_pad, d_feat), jnp.float32),
          pltpu.VMEM((n_group * K,), jnp.int32),
          pltpu.VMEM((n_group, K), jnp.int32),
          pltpu.VMEM((n_group * K,), jnp.float32),
          [pltpu.VMEM((K, d_feat), jnp.float32)] * NBUF,
          [pltpu.SemaphoreType.DMA] * NBUF,
          [pltpu.SemaphoreType.DMA] * NBUF,
      ],
  )
  def sc_kernel(x_hbm, src_hbm, dst_hbm, val_hbm, part_hbm,
                acc, src_v, dst_v, val_v, bufs, gsems, ssems):
    c = lax.axis_index("c")
    s = lax.axis_index("s")
    wid = s * NC + c

    def stage_idx(stage):
      widx = wid * N_STAGES + stage
      gk = n_group * K
      pltpu.sync_copy(src_hbm.at[pl.ds(widx * gk, gk)], src_v)
      pltpu.sync_copy(dst_hbm.at[widx], dst_v)
      pltpu.sync_copy(val_hbm.at[pl.ds(widx * gk, gk)], val_v)

    def start_gather(j, b):
      pltpu.async_copy(x_hbm.at[src_v.at[pl.ds(j * K, K)]], bufs[b], gsems[b])

    def wait_gather(j, b):
      pltpu.make_async_copy(
          x_hbm.at[src_v.at[pl.ds(j * K, K)]], bufs[b], gsems[b]).wait()

    def start_scatter(j, b):
      pltpu.async_copy(bufs[b], acc.at[dst_v.at[j]], ssems[b], add=True)

    def wait_scatter(j, b):
      pltpu.make_async_copy(bufs[b], acc.at[dst_v.at[j]], ssems[b]).wait()

    def scale(j, b):
      jbase = jnp.full((LANES,), j * K, jnp.int32)
      buf = bufs[b]

      @plsc.parallel_loop(0, K, unroll=4)
      def _(e):
        vsplat = plsc.load_gather(
            val_v, [jbase + jnp.full((LANES,), e, jnp.int32)])
        for t in range(d_feat // LANES):
          sl = buf.at[e, pl.ds(t * LANES, LANES)]
          sl[...] = sl[...] * vsplat

    @pl.loop(0, K)
    def _(i):
      for t in range(d_feat // LANES):
        bufs[0].at[i, pl.ds(t * LANES, LANES)][...] = jnp.zeros(
            (LANES,), jnp.float32)

    for off, sz in copies:
      pltpu.sync_copy(
          bufs[0].at[pl.ds(0, sz)],
          acc.at[pl.ds(s * rows_per_tile + off, sz)])

    for stage in range(N_STAGES):
      stage_idx(stage)
      start_gather(0, 0)
      start_gather(1, 1)
      if stage == 0:
        plsc.subcore_barrier()

      @pl.loop(0, n_tri)
      def _(t):
        for b in range(NBUF):
          j = NBUF * t + b
          bp = (b + NBUF - 1) % NBUF

          def drain_and_prefetch(j=j, bp=bp, first=(b == 0)):
            if first:
              @pl.when(t >= 1)
              def _():
                wait_scatter(j - 1, bp)
            else:
              wait_scatter(j - 1, bp)

          drain_and_prefetch()
          if b == 0:
            start_gather(j + 2, bp)
          else:
            @pl.when(t < n_tri - 1)
            def _():
              start_gather(j + 2, bp)
          wait_gather(j, b)
          scale(j, b)
          start_scatter(j, b)

      wait_scatter(n_group - 1, (n_group - 1) % NBUF)

    plsc.subcore_barrier()

    for off, sz in copies:
      r0 = s * rows_per_tile + off
      pltpu.sync_copy(acc.at[pl.ds(r0, sz)],
                      part_hbm.at[c, pl.ds(r0, sz)])

  return sc_kernel(x, src, dst, val)


def _tc_merge(partials, n_nodes, d_feat):
  def body(p_ref, o_ref):
    o_ref[...] = p_ref[0, :n_nodes] + p_ref[1, :n_nodes]

  return pl.pallas_call(
      body,
      out_shape=jax.ShapeDtypeStruct((n_nodes, d_feat), jnp.float32),
  )(partials)


@jax.jit
def _run(x, adj_indices, adj_values):
  n_nodes, d_feat = x.shape
  n_edges = adj_values.shape[0]
  dst = adj_indices[0].astype(jnp.int32)
  src = adj_indices[1].astype(jnp.int32)
  val = adj_values.astype(jnp.float32)

  m = N_STAGES * NBUF
  n_chunks = -(-n_edges // (NW * K * m)) * m
  e_pad = NW * n_chunks * K
  pad = e_pad - n_edges
  acc_spare = _acc_rows(n_nodes) - n_nodes
  if acc_spare > 0:
    pad_dst = jnp.asarray(
        n_nodes + (np.arange(pad) % acc_spare), dtype=jnp.int32)
  else:
    pad_dst = jnp.asarray(np.arange(pad) % n_nodes, dtype=jnp.int32)
  pad_src = jnp.asarray(np.arange(pad) % n_nodes, dtype=jnp.int32)

  src = jnp.concatenate([src, pad_src])
  dst = jnp.concatenate([dst, pad_dst]).reshape(
      NW * N_STAGES, n_chunks // N_STAGES, K)
  val = jnp.pad(val, (0, pad))

  partials = _sc_partials(x, src, dst, val, n_chunks, n_nodes, d_feat)
  return _tc_merge(partials, n_nodes, d_feat)


def kernel(x, adj_indices, adj_values):
  return _run(x, adj_indices, adj_values)
